# baseline (speedup 1.0000x reference)
import jax
import jax.numpy as jnp
from jax import lax
from jax.experimental import pallas as pl
from jax.experimental.pallas import tpu as pltpu
from jax.experimental.pallas import tpu_sc as plsc

N_NODES = 10000
N_EDGES = 320000
D = 128
N_GRAPHS = 64

NC, NS = 2, 16
NW = NC * NS
CHUNK = 64
BLK = 4
TOT_CH = 5120
CPW0 = 200
CPW1 = TOT_CH // NS - CPW0
E_PAD = TOT_CH * CHUNK
N_PAD = 10240
DUMMY_ROW = 10016
RPT = N_PAD // NS
RB = RPT // CHUNK

ROW_BLK = 512
N_BLKS = N_PAD // ROW_BLK


def _mesh():
    return plsc.VectorSubcoreMesh(core_axis_name="c", subcore_axis_name="s",
                                  num_cores=NC, num_subcores=NS)


def _make_edge_pass(with_cnt):
    out_types = [jax.ShapeDtypeStruct((NC, N_PAD, D), jnp.float32)]
    scratch = [
        pltpu.VMEM((BLK, CHUNK), jnp.int32),
        pltpu.VMEM((BLK, CHUNK), jnp.int32),
        pltpu.VMEM((BLK, CHUNK), jnp.int32),
        pltpu.VMEM((BLK, CHUNK), jnp.int32),
        pltpu.VMEM((CHUNK, D), jnp.float32),
        pltpu.VMEM((CHUNK, D), jnp.float32),
        pltpu.VMEM((CHUNK, D), jnp.float32),
        pltpu.VMEM((CHUNK, D), jnp.float32),
        pltpu.VMEM_SHARED((N_PAD, D), jnp.float32),
    ] + [pltpu.SemaphoreType.DMA] * 12
    if with_cnt:
        out_types.append(jax.ShapeDtypeStruct((NC, N_PAD), jnp.float32))
        scratch += [pltpu.VMEM((CHUNK,), jnp.float32),
                    pltpu.VMEM((RPT,), jnp.float32),
                    pltpu.VMEM_SHARED((N_PAD,), jnp.float32)] + [pltpu.SemaphoreType.DMA] * 4

    def body(*refs):
        if with_cnt:
            (table, src2, dst2, zrows, out, deg_out,
             is0, is1, id0, id1, b0, b1, b2, b3, acc,
             g0, g1, g2, g3, s0, s1, s2, s3, i0, i1, x0, x1,
             cbuf, degv, cnt, c0, c1, c2, c3) = refs
        else:
            (table, src2, dst2, zrows, out,
             is0, is1, id0, id1, b0, b1, b2, b3, acc,
             g0, g1, g2, g3, s0, s1, s2, s3, i0, i1, x0, x1) = refs
            deg_out = cbuf = degv = cnt = c0 = c1 = c2 = c3 = None
        bufs = (b0, b1, b2, b3)
        isb = (is0, is1)
        idb = (id0, id1)
        gsem = (g0, g1, g2, g3)
        ssem = (s0, s1, s2, s3)
        csem = (c0, c1, c2, c3)
        isem = (i0, i1)
        c = lax.axis_index("c")
        s = lax.axis_index("s")
        brow0 = jnp.where(c == 0, s * CPW0, NS * CPW0 + s * CPW1)
        nblk2 = jnp.where(c == 0, (CPW0 // BLK) // 2, (CPW1 // BLK) // 2)

        def idx_issue(parity, blkid):
            r = brow0 + blkid * BLK
            pltpu.async_copy(src2.at[pl.ds(r, BLK)], isb[parity],
                             isem[parity])
            pltpu.async_copy(dst2.at[pl.ds(r, BLK)], idb[parity],
                             isem[parity])

        def idx_wait(parity):
            pltpu.make_async_copy(src2.at[pl.ds(brow0, BLK)], isb[parity],
                                  isem[parity]).wait()
            pltpu.make_async_copy(dst2.at[pl.ds(brow0, BLK)], idb[parity],
                                  isem[parity]).wait()

        idx_issue(0, 0)
        pltpu.sync_copy(zrows, b0)
        if with_cnt:
            zero16 = jnp.zeros((16,), jnp.float32)
            for i in range(CHUNK // 16):
                cbuf[pl.ds(i * 16, 16)] = zero16
        for k in range(RB):
            r0 = s * RPT + k * CHUNK
            pltpu.sync_copy(b0, acc.at[pl.ds(r0, CHUNK)])
            if with_cnt:
                pltpu.sync_copy(cbuf, cnt.at[pl.ds(r0, CHUNK)])
        if with_cnt:
            one16 = jnp.ones((16,), jnp.float32)
            for i in range(CHUNK // 16):
                cbuf[pl.ds(i * 16, 16)] = one16
        plsc.subcore_barrier()

        def drain(parity):
            for b in range(BLK):
                pltpu.make_async_copy(bufs[b], acc.at[idb[parity].at[b]],
                                      ssem[b]).wait()
                if with_cnt:
                    pltpu.make_async_copy(cbuf, cnt.at[idb[parity].at[b]],
                                          csem[b]).wait()

        def halfblock(jj, parity, first):
            idx_wait(parity)
            if first:
                @pl.when(jj > 0)
                def _():
                    drain(parity)
            else:
                drain(parity)
            nxt = 2 * jj + (1 if parity == 0 else 2)
            if parity == 0:
                idx_issue(1, nxt)
            else:
                @pl.when(jj < nblk2 - 1)
                def _():
                    idx_issue(0, nxt)
            for b in range(BLK):
                pltpu.async_copy(table.at[isb[parity].at[b]], bufs[b],
                                 gsem[b])
            for b in range(BLK):
                pltpu.make_async_copy(table.at[isb[parity].at[b]], bufs[b],
                                      gsem[b]).wait()
                pltpu.async_copy(bufs[b], acc.at[idb[parity].at[b]],
                                 ssem[b], add=True)
                if with_cnt:
                    pltpu.async_copy(cbuf, cnt.at[idb[parity].at[b]],
                                     csem[b], add=True)

        def step(jj, carry):
            halfblock(jj, 0, True)
            halfblock(jj, 1, False)
            return carry

        lax.fori_loop(0, nblk2, step, 0)
        for b in range(BLK):
            pltpu.make_async_copy(bufs[b], acc.at[idb[0].at[b]],
                                  ssem[b]).wait()
            if with_cnt:
                pltpu.make_async_copy(cbuf, cnt.at[idb[0].at[b]],
                                      csem[b]).wait()
        plsc.subcore_barrier()

        for k in range(RB):
            r0 = s * RPT + k * CHUNK
            pltpu.sync_copy(acc.at[pl.ds(r0, CHUNK)], b0)
            pltpu.sync_copy(b0, out.at[c, pl.ds(r0, CHUNK)])
        if with_cnt:
            pltpu.sync_copy(cnt.at[pl.ds(s * RPT, RPT)], degv)
            pltpu.sync_copy(degv, deg_out.at[c, pl.ds(s * RPT, RPT)])

    return pl.kernel(body, out_type=tuple(out_types), mesh=_mesh(),
                     scratch_types=scratch)


def _mm2_body(x_ref, wl_ref, wr_ref, b_ref, y_ref, r_ref):
    xb = x_ref[...]
    y_ref[...] = jnp.dot(xb, wl_ref[...], preferred_element_type=jnp.float32)
    r_ref[...] = (jnp.dot(xb, wr_ref[...], preferred_element_type=jnp.float32)
                  + b_ref[...])


def _stage1_body(p1_ref, degm_ref, r1_ref, b3_ref, w2l_ref,
                 z_ref, ph1_ref):
    i = pl.program_id(0)
    p = p1_ref[0] + p1_ref[1]
    h = jnp.maximum(p / degm_ref[...] + r1_ref[...], 0.0)
    z_ref[...] = jnp.dot(h, w2l_ref[...], preferred_element_type=jnp.float32)
    bb = b3_ref[0]
    ohT = (bb == lax.broadcasted_iota(jnp.int32, (N_GRAPHS, ROW_BLK), 0)
           ).astype(jnp.float32)

    @pl.when(i == 0)
    def _():
        ph1_ref[...] = jnp.zeros_like(ph1_ref)

    ph1_ref[...] += jnp.dot(ohT, h, preferred_element_type=jnp.float32)


def _stage2_body(p2_ref, degm_ref, b3_ref, ph1_ref, w2r_ref, b2_ref,
                 wfc_ref, bfc_ref, out_ref, s_scr, cb_scr):
    i = pl.program_id(0)
    p = p2_ref[0] + p2_ref[1]
    aggm = p / degm_ref[...]
    bb = b3_ref[0]
    ohT = (bb == lax.broadcasted_iota(jnp.int32, (N_GRAPHS, ROW_BLK), 0)
           ).astype(jnp.float32)

    @pl.when(i == 0)
    def _():
        s_scr[...] = jnp.zeros_like(s_scr)
        cb_scr[...] = jnp.zeros_like(cb_scr)

    s_scr[...] += jnp.dot(ohT, aggm, preferred_element_type=jnp.float32)
    cb_scr[...] += jnp.dot(ohT, jnp.ones((ROW_BLK, D), jnp.float32),
                           preferred_element_type=jnp.float32)

    @pl.when(i == N_BLKS - 1)
    def _():
        cb = jnp.maximum(cb_scr[:, 0:1], 1.0)
        g = (s_scr[...] / cb
             + jnp.dot(ph1_ref[...] / cb, w2r_ref[...],
                       preferred_element_type=jnp.float32)
             + b2_ref[...])
        out_ref[...] = (jnp.dot(g, wfc_ref[...],
                                preferred_element_type=jnp.float32)
                        + bfc_ref[...])


def kernel(x, edge_index, batch, W1l, b1, W1r, W2l, b2, W2r, Wfc, bfc):
    src = edge_index[0].astype(jnp.int32)
    dst = edge_index[1].astype(jnp.int32)
    epad = E_PAD - N_EDGES
    src_p = jnp.concatenate([src, jnp.zeros((epad,), jnp.int32)]
                            ).reshape(E_PAD // CHUNK, CHUNK)
    dst_p = jnp.concatenate([dst, jnp.full((epad,), DUMMY_ROW, jnp.int32)]
                            ).reshape(E_PAD // CHUNK, CHUNK)
    npad = N_PAD - N_NODES
    x_p = jnp.concatenate([x, jnp.zeros((npad, D), x.dtype)])
    batch3 = jnp.concatenate([batch.astype(jnp.int32),
                              jnp.full((npad,), N_GRAPHS, jnp.int32)]
                             ).reshape(N_BLKS, 1, ROW_BLK)
    zrows = jnp.zeros((CHUNK, D), jnp.float32)

    full = pl.BlockSpec((D, D), lambda i: (0, 0))
    rowb = pl.BlockSpec((ROW_BLK, D), lambda i: (i, 0))

    y1, r1 = pl.pallas_call(
        _mm2_body,
        grid=(N_BLKS,),
        in_specs=[rowb, full, full, pl.BlockSpec((1, D), lambda i: (0, 0))],
        out_specs=[rowb, rowb],
        out_shape=[jax.ShapeDtypeStruct((N_PAD, D), jnp.float32)] * 2,
    )(x_p, W1l, W1r, b1.reshape(1, D))

    p1, deg = _make_edge_pass(True)(y1, src_p, dst_p, zrows)
    degm = jnp.maximum(deg[0] + deg[1], 1.0).reshape(N_PAD, 1)

    pblk = pl.BlockSpec((NC, ROW_BLK, D), lambda i: (0, i, 0))
    dblk = pl.BlockSpec((ROW_BLK, 1), lambda i: (i, 0))
    bblk = pl.BlockSpec((1, 1, ROW_BLK), lambda i: (i, 0, 0))
    gfull = pl.BlockSpec((N_GRAPHS, D), lambda i: (0, 0))

    z, ph1 = pl.pallas_call(
        _stage1_body,
        grid=(N_BLKS,),
        in_specs=[pblk, dblk, rowb, bblk, full],
        out_specs=[rowb, gfull],
        out_shape=[jax.ShapeDtypeStruct((N_PAD, D), jnp.float32),
                   jax.ShapeDtypeStruct((N_GRAPHS, D), jnp.float32)],
    )(p1, degm, r1, batch3, W2l)

    (p2,) = _make_edge_pass(False)(z, src_p, dst_p, zrows)

    out = pl.pallas_call(
        _stage2_body,
        grid=(N_BLKS,),
        in_specs=[pblk, dblk, bblk, gfull, full,
                  pl.BlockSpec((1, D), lambda i: (0, 0)),
                  pl.BlockSpec((D, 4), lambda i: (0, 0)),
                  pl.BlockSpec((1, 4), lambda i: (0, 0))],
        out_specs=pl.BlockSpec((N_GRAPHS, 4), lambda i: (0, 0)),
        out_shape=jax.ShapeDtypeStruct((N_GRAPHS, 4), jnp.float32),
        scratch_shapes=[pltpu.VMEM((N_GRAPHS, D), jnp.float32),
                        pltpu.VMEM((N_GRAPHS, D), jnp.float32)],
    )(p2, degm, batch3, ph1, W2r, b2.reshape(1, D), Wfc, bfc.reshape(1, 4))

    return out


# speedup vs baseline: 1.0885x; 1.0885x over previous
import jax
import jax.numpy as jnp
from jax import lax
from jax.experimental import pallas as pl
from jax.experimental.pallas import tpu as pltpu
from jax.experimental.pallas import tpu_sc as plsc

N_NODES = 10000
N_EDGES = 320000
D = 128
N_GRAPHS = 64

NC, NS = 2, 16
NW = NC * NS
CHUNK = 64
BLK = 4
TOT_CH = 5120
CPW0 = 256
CPW1 = TOT_CH // NS - CPW0
E_PAD = TOT_CH * CHUNK
N_PAD = 10240
DUMMY_ROW = 10016
RPT = N_PAD // NS
RB = RPT // CHUNK

ROW_BLK = 512
N_BLKS = N_PAD // ROW_BLK


def _mesh():
    return plsc.VectorSubcoreMesh(core_axis_name="c", subcore_axis_name="s",
                                  num_cores=NC, num_subcores=NS)


def _make_edge_pass(with_cnt):
    out_types = [jax.ShapeDtypeStruct((NC, N_PAD, D), jnp.float32)]
    scratch = [
        pltpu.VMEM((BLK, CHUNK), jnp.int32),
        pltpu.VMEM((BLK, CHUNK), jnp.int32),
        pltpu.VMEM((BLK, CHUNK), jnp.int32),
        pltpu.VMEM((BLK, CHUNK), jnp.int32),
        pltpu.VMEM((CHUNK, D), jnp.float32),
        pltpu.VMEM((CHUNK, D), jnp.float32),
        pltpu.VMEM((CHUNK, D), jnp.float32),
        pltpu.VMEM((CHUNK, D), jnp.float32),
        pltpu.VMEM_SHARED((N_PAD, D), jnp.float32),
    ] + [pltpu.SemaphoreType.DMA] * 12
    if with_cnt:
        out_types.append(jax.ShapeDtypeStruct((NC, N_PAD), jnp.float32))
        scratch += [pltpu.VMEM((CHUNK,), jnp.float32),
                    pltpu.VMEM((RPT,), jnp.float32),
                    pltpu.VMEM_SHARED((N_PAD,), jnp.float32)] + [pltpu.SemaphoreType.DMA] * 4

    def body(*refs):
        if with_cnt:
            (table, src2, dst2, zrows, out, deg_out,
             is0, is1, id0, id1, b0, b1, b2, b3, acc,
             g0, g1, g2, g3, s0, s1, s2, s3, i0, i1, x0, x1,
             cbuf, degv, cnt, c0, c1, c2, c3) = refs
        else:
            (table, src2, dst2, zrows, out,
             is0, is1, id0, id1, b0, b1, b2, b3, acc,
             g0, g1, g2, g3, s0, s1, s2, s3, i0, i1, x0, x1) = refs
            deg_out = cbuf = degv = cnt = c0 = c1 = c2 = c3 = None
        bufs = (b0, b1, b2, b3)
        isb = (is0, is1)
        idb = (id0, id1)
        gsem = (g0, g1, g2, g3)
        ssem = (s0, s1, s2, s3)
        csem = (c0, c1, c2, c3)
        isem = (i0, i1)
        c = lax.axis_index("c")
        s = lax.axis_index("s")
        brow0 = jnp.where(c == 0, s * CPW0, NS * CPW0 + s * CPW1)
        nblk2 = jnp.where(c == 0, (CPW0 // BLK) // 2, (CPW1 // BLK) // 2)

        def idx_issue(parity, blkid):
            r = brow0 + blkid * BLK
            pltpu.async_copy(src2.at[pl.ds(r, BLK)], isb[parity],
                             isem[parity])
            pltpu.async_copy(dst2.at[pl.ds(r, BLK)], idb[parity],
                             isem[parity])

        def idx_wait(parity):
            pltpu.make_async_copy(src2.at[pl.ds(brow0, BLK)], isb[parity],
                                  isem[parity]).wait()
            pltpu.make_async_copy(dst2.at[pl.ds(brow0, BLK)], idb[parity],
                                  isem[parity]).wait()

        idx_issue(0, 0)
        pltpu.sync_copy(zrows, b0)
        if with_cnt:
            zero16 = jnp.zeros((16,), jnp.float32)
            for i in range(CHUNK // 16):
                cbuf[pl.ds(i * 16, 16)] = zero16
        for k in range(RB):
            r0 = s * RPT + k * CHUNK
            pltpu.sync_copy(b0, acc.at[pl.ds(r0, CHUNK)])
            if with_cnt:
                pltpu.sync_copy(cbuf, cnt.at[pl.ds(r0, CHUNK)])
        if with_cnt:
            one16 = jnp.ones((16,), jnp.float32)
            for i in range(CHUNK // 16):
                cbuf[pl.ds(i * 16, 16)] = one16
        plsc.subcore_barrier()

        def drain(parity):
            for b in range(BLK):
                pltpu.make_async_copy(bufs[b], acc.at[idb[parity].at[b]],
                                      ssem[b]).wait()
                if with_cnt:
                    pltpu.make_async_copy(cbuf, cnt.at[idb[parity].at[b]],
                                          csem[b]).wait()

        def halfblock(jj, parity, first):
            idx_wait(parity)
            if first:
                @pl.when(jj > 0)
                def _():
                    drain(parity)
            else:
                drain(parity)
            nxt = 2 * jj + (1 if parity == 0 else 2)
            if parity == 0:
                idx_issue(1, nxt)
            else:
                @pl.when(jj < nblk2 - 1)
                def _():
                    idx_issue(0, nxt)
            for b in range(BLK):
                pltpu.async_copy(table.at[isb[parity].at[b]], bufs[b],
                                 gsem[b])
            for b in range(BLK):
                pltpu.make_async_copy(table.at[isb[parity].at[b]], bufs[b],
                                      gsem[b]).wait()
                pltpu.async_copy(bufs[b], acc.at[idb[parity].at[b]],
                                 ssem[b], add=True)
                if with_cnt:
                    pltpu.async_copy(cbuf, cnt.at[idb[parity].at[b]],
                                     csem[b], add=True)

        def step(jj, carry):
            halfblock(jj, 0, True)
            halfblock(jj, 1, False)
            return carry

        lax.fori_loop(0, nblk2, step, 0)
        for b in range(BLK):
            pltpu.make_async_copy(bufs[b], acc.at[idb[0].at[b]],
                                  ssem[b]).wait()
            if with_cnt:
                pltpu.make_async_copy(cbuf, cnt.at[idb[0].at[b]],
                                      csem[b]).wait()
        plsc.subcore_barrier()

        for k in range(RB):
            r0 = s * RPT + k * CHUNK
            pltpu.sync_copy(acc.at[pl.ds(r0, CHUNK)], b0)
            pltpu.sync_copy(b0, out.at[c, pl.ds(r0, CHUNK)])
        if with_cnt:
            pltpu.sync_copy(cnt.at[pl.ds(s * RPT, RPT)], degv)
            pltpu.sync_copy(degv, deg_out.at[c, pl.ds(s * RPT, RPT)])

    return pl.kernel(body, out_type=tuple(out_types), mesh=_mesh(),
                     scratch_types=scratch)


def _mm2_body(x_ref, wl_ref, wr_ref, b_ref, y_ref, r_ref):
    xb = x_ref[...]
    y_ref[...] = jnp.dot(xb, wl_ref[...], preferred_element_type=jnp.float32)
    r_ref[...] = (jnp.dot(xb, wr_ref[...], preferred_element_type=jnp.float32)
                  + b_ref[...])


def _stage1_body(p1_ref, degm_ref, r1_ref, b3_ref, w2l_ref,
                 z_ref, ph1_ref):
    i = pl.program_id(0)
    p = p1_ref[0] + p1_ref[1]
    h = jnp.maximum(p / degm_ref[...] + r1_ref[...], 0.0)
    z_ref[...] = jnp.dot(h, w2l_ref[...], preferred_element_type=jnp.float32)
    bb = b3_ref[0]
    ohT = (bb == lax.broadcasted_iota(jnp.int32, (N_GRAPHS, ROW_BLK), 0)
           ).astype(jnp.float32)

    @pl.when(i == 0)
    def _():
        ph1_ref[...] = jnp.zeros_like(ph1_ref)

    ph1_ref[...] += jnp.dot(ohT, h, preferred_element_type=jnp.float32)


def _stage2_body(p2_ref, degm_ref, b3_ref, ph1_ref, w2r_ref, b2_ref,
                 wfc_ref, bfc_ref, out_ref, s_scr, cb_scr):
    i = pl.program_id(0)
    p = p2_ref[0] + p2_ref[1]
    aggm = p / degm_ref[...]
    bb = b3_ref[0]
    ohT = (bb == lax.broadcasted_iota(jnp.int32, (N_GRAPHS, ROW_BLK), 0)
           ).astype(jnp.float32)

    @pl.when(i == 0)
    def _():
        s_scr[...] = jnp.zeros_like(s_scr)
        cb_scr[...] = jnp.zeros_like(cb_scr)

    s_scr[...] += jnp.dot(ohT, aggm, preferred_element_type=jnp.float32)
    cb_scr[...] += jnp.dot(ohT, jnp.ones((ROW_BLK, D), jnp.float32),
                           preferred_element_type=jnp.float32)

    @pl.when(i == N_BLKS - 1)
    def _():
        cb = jnp.maximum(cb_scr[:, 0:1], 1.0)
        g = (s_scr[...] / cb
             + jnp.dot(ph1_ref[...] / cb, w2r_ref[...],
                       preferred_element_type=jnp.float32)
             + b2_ref[...])
        out_ref[...] = (jnp.dot(g, wfc_ref[...],
                                preferred_element_type=jnp.float32)
                        + bfc_ref[...])


def kernel(x, edge_index, batch, W1l, b1, W1r, W2l, b2, W2r, Wfc, bfc):
    src = edge_index[0].astype(jnp.int32)
    dst = edge_index[1].astype(jnp.int32)
    epad = E_PAD - N_EDGES
    src_p = jnp.concatenate([src, jnp.zeros((epad,), jnp.int32)]
                            ).reshape(E_PAD // CHUNK, CHUNK)
    dst_p = jnp.concatenate([dst, jnp.full((epad,), DUMMY_ROW, jnp.int32)]
                            ).reshape(E_PAD // CHUNK, CHUNK)
    npad = N_PAD - N_NODES
    x_p = jnp.concatenate([x, jnp.zeros((npad, D), x.dtype)])
    batch3 = jnp.concatenate([batch.astype(jnp.int32),
                              jnp.full((npad,), N_GRAPHS, jnp.int32)]
                             ).reshape(N_BLKS, 1, ROW_BLK)
    zrows = jnp.zeros((CHUNK, D), jnp.float32)

    full = pl.BlockSpec((D, D), lambda i: (0, 0))
    rowb = pl.BlockSpec((ROW_BLK, D), lambda i: (i, 0))

    y1, r1 = pl.pallas_call(
        _mm2_body,
        grid=(N_BLKS,),
        in_specs=[rowb, full, full, pl.BlockSpec((1, D), lambda i: (0, 0))],
        out_specs=[rowb, rowb],
        out_shape=[jax.ShapeDtypeStruct((N_PAD, D), jnp.float32)] * 2,
    )(x_p, W1l, W1r, b1.reshape(1, D))

    p1, deg = _make_edge_pass(True)(y1, src_p, dst_p, zrows)
    degm = jnp.maximum(deg[0] + deg[1], 1.0).reshape(N_PAD, 1)

    pblk = pl.BlockSpec((NC, ROW_BLK, D), lambda i: (0, i, 0))
    dblk = pl.BlockSpec((ROW_BLK, 1), lambda i: (i, 0))
    bblk = pl.BlockSpec((1, 1, ROW_BLK), lambda i: (i, 0, 0))
    gfull = pl.BlockSpec((N_GRAPHS, D), lambda i: (0, 0))

    z, ph1 = pl.pallas_call(
        _stage1_body,
        grid=(N_BLKS,),
        in_specs=[pblk, dblk, rowb, bblk, full],
        out_specs=[rowb, gfull],
        out_shape=[jax.ShapeDtypeStruct((N_PAD, D), jnp.float32),
                   jax.ShapeDtypeStruct((N_GRAPHS, D), jnp.float32)],
    )(p1, degm, r1, batch3, W2l)

    (p2,) = _make_edge_pass(False)(z, src_p, dst_p, zrows)

    out = pl.pallas_call(
        _stage2_body,
        grid=(N_BLKS,),
        in_specs=[pblk, dblk, bblk, gfull, full,
                  pl.BlockSpec((1, D), lambda i: (0, 0)),
                  pl.BlockSpec((D, 4), lambda i: (0, 0)),
                  pl.BlockSpec((1, 4), lambda i: (0, 0))],
        out_specs=pl.BlockSpec((N_GRAPHS, 4), lambda i: (0, 0)),
        out_shape=jax.ShapeDtypeStruct((N_GRAPHS, 4), jnp.float32),
        scratch_shapes=[pltpu.VMEM((N_GRAPHS, D), jnp.float32),
                        pltpu.VMEM((N_GRAPHS, D), jnp.float32)],
    )(p2, degm, batch3, ph1, W2r, b2.reshape(1, D), Wfc, bfc.reshape(1, 4))

    return out


# speedup vs baseline: 1.1393x; 1.0466x over previous
import jax
import jax.numpy as jnp
from jax import lax
from jax.experimental import pallas as pl
from jax.experimental.pallas import tpu as pltpu
from jax.experimental.pallas import tpu_sc as plsc

N_NODES = 10000
N_EDGES = 320000
D = 128
N_GRAPHS = 64

NC, NS = 2, 16
NW = NC * NS
CHUNK = 64
BLK = 4
TOT_CH = 5120
CPW0 = 288
CPW1 = TOT_CH // NS - CPW0
E_PAD = TOT_CH * CHUNK
N_PAD = 10240
DUMMY_ROW = 10016
RPT = N_PAD // NS
RB = RPT // CHUNK

ROW_BLK = 512
N_BLKS = N_PAD // ROW_BLK


def _mesh():
    return plsc.VectorSubcoreMesh(core_axis_name="c", subcore_axis_name="s",
                                  num_cores=NC, num_subcores=NS)


def _make_edge_pass(with_cnt):
    out_types = [jax.ShapeDtypeStruct((NC, N_PAD, D), jnp.float32)]
    scratch = [
        pltpu.VMEM((BLK, CHUNK), jnp.int32),
        pltpu.VMEM((BLK, CHUNK), jnp.int32),
        pltpu.VMEM((BLK, CHUNK), jnp.int32),
        pltpu.VMEM((BLK, CHUNK), jnp.int32),
        pltpu.VMEM((CHUNK, D), jnp.float32),
        pltpu.VMEM((CHUNK, D), jnp.float32),
        pltpu.VMEM((CHUNK, D), jnp.float32),
        pltpu.VMEM((CHUNK, D), jnp.float32),
        pltpu.VMEM_SHARED((N_PAD, D), jnp.float32),
    ] + [pltpu.SemaphoreType.DMA] * 12
    if with_cnt:
        out_types.append(jax.ShapeDtypeStruct((NC, N_PAD), jnp.float32))
        scratch += [pltpu.VMEM((CHUNK,), jnp.float32),
                    pltpu.VMEM((RPT,), jnp.float32),
                    pltpu.VMEM_SHARED((N_PAD,), jnp.float32)] + [pltpu.SemaphoreType.DMA] * 4

    def body(*refs):
        if with_cnt:
            (table, src2, dst2, zrows, out, deg_out,
             is0, is1, id0, id1, b0, b1, b2, b3, acc,
             g0, g1, g2, g3, s0, s1, s2, s3, i0, i1, x0, x1,
             cbuf, degv, cnt, c0, c1, c2, c3) = refs
        else:
            (table, src2, dst2, zrows, out,
             is0, is1, id0, id1, b0, b1, b2, b3, acc,
             g0, g1, g2, g3, s0, s1, s2, s3, i0, i1, x0, x1) = refs
            deg_out = cbuf = degv = cnt = c0 = c1 = c2 = c3 = None
        bufs = (b0, b1, b2, b3)
        isb = (is0, is1)
        idb = (id0, id1)
        gsem = (g0, g1, g2, g3)
        ssem = (s0, s1, s2, s3)
        csem = (c0, c1, c2, c3)
        isem = (i0, i1)
        c = lax.axis_index("c")
        s = lax.axis_index("s")
        brow0 = jnp.where(c == 0, s * CPW0, NS * CPW0 + s * CPW1)
        nblk2 = jnp.where(c == 0, (CPW0 // BLK) // 2, (CPW1 // BLK) // 2)

        def idx_issue(parity, blkid):
            r = brow0 + blkid * BLK
            pltpu.async_copy(src2.at[pl.ds(r, BLK)], isb[parity],
                             isem[parity])
            pltpu.async_copy(dst2.at[pl.ds(r, BLK)], idb[parity],
                             isem[parity])

        def idx_wait(parity):
            pltpu.make_async_copy(src2.at[pl.ds(brow0, BLK)], isb[parity],
                                  isem[parity]).wait()
            pltpu.make_async_copy(dst2.at[pl.ds(brow0, BLK)], idb[parity],
                                  isem[parity]).wait()

        idx_issue(0, 0)
        pltpu.sync_copy(zrows, b0)
        if with_cnt:
            zero16 = jnp.zeros((16,), jnp.float32)
            for i in range(CHUNK // 16):
                cbuf[pl.ds(i * 16, 16)] = zero16
        for k in range(RB):
            r0 = s * RPT + k * CHUNK
            pltpu.sync_copy(b0, acc.at[pl.ds(r0, CHUNK)])
            if with_cnt:
                pltpu.sync_copy(cbuf, cnt.at[pl.ds(r0, CHUNK)])
        if with_cnt:
            one16 = jnp.ones((16,), jnp.float32)
            for i in range(CHUNK // 16):
                cbuf[pl.ds(i * 16, 16)] = one16
        plsc.subcore_barrier()

        def drain(parity):
            for b in range(BLK):
                pltpu.make_async_copy(bufs[b], acc.at[idb[parity].at[b]],
                                      ssem[b]).wait()
                if with_cnt:
                    pltpu.make_async_copy(cbuf, cnt.at[idb[parity].at[b]],
                                          csem[b]).wait()

        def halfblock(jj, parity, first):
            idx_wait(parity)
            if first:
                @pl.when(jj > 0)
                def _():
                    drain(parity)
            else:
                drain(parity)
            nxt = 2 * jj + (1 if parity == 0 else 2)
            if parity == 0:
                idx_issue(1, nxt)
            else:
                @pl.when(jj < nblk2 - 1)
                def _():
                    idx_issue(0, nxt)
            for b in range(BLK):
                pltpu.async_copy(table.at[isb[parity].at[b]], bufs[b],
                                 gsem[b])
            for b in range(BLK):
                pltpu.make_async_copy(table.at[isb[parity].at[b]], bufs[b],
                                      gsem[b]).wait()
                pltpu.async_copy(bufs[b], acc.at[idb[parity].at[b]],
                                 ssem[b], add=True)
                if with_cnt:
                    pltpu.async_copy(cbuf, cnt.at[idb[parity].at[b]],
                                     csem[b], add=True)

        def step(jj, carry):
            halfblock(jj, 0, True)
            halfblock(jj, 1, False)
            return carry

        lax.fori_loop(0, nblk2, step, 0)
        for b in range(BLK):
            pltpu.make_async_copy(bufs[b], acc.at[idb[0].at[b]],
                                  ssem[b]).wait()
            if with_cnt:
                pltpu.make_async_copy(cbuf, cnt.at[idb[0].at[b]],
                                      csem[b]).wait()
        plsc.subcore_barrier()

        for k in range(RB):
            r0 = s * RPT + k * CHUNK
            pltpu.sync_copy(acc.at[pl.ds(r0, CHUNK)], b0)
            pltpu.sync_copy(b0, out.at[c, pl.ds(r0, CHUNK)])
        if with_cnt:
            pltpu.sync_copy(cnt.at[pl.ds(s * RPT, RPT)], degv)
            pltpu.sync_copy(degv, deg_out.at[c, pl.ds(s * RPT, RPT)])

    return pl.kernel(body, out_type=tuple(out_types), mesh=_mesh(),
                     scratch_types=scratch)


def _mm2_body(x_ref, wl_ref, wr_ref, b_ref, y_ref, r_ref):
    xb = x_ref[...]
    y_ref[...] = jnp.dot(xb, wl_ref[...], preferred_element_type=jnp.float32)
    r_ref[...] = (jnp.dot(xb, wr_ref[...], preferred_element_type=jnp.float32)
                  + b_ref[...])


def _stage1_body(p1_ref, degm_ref, r1_ref, b3_ref, w2l_ref,
                 z_ref, ph1_ref):
    i = pl.program_id(0)
    p = p1_ref[0] + p1_ref[1]
    h = jnp.maximum(p / degm_ref[...] + r1_ref[...], 0.0)
    z_ref[...] = jnp.dot(h, w2l_ref[...], preferred_element_type=jnp.float32)
    bb = b3_ref[0]
    ohT = (bb == lax.broadcasted_iota(jnp.int32, (N_GRAPHS, ROW_BLK), 0)
           ).astype(jnp.float32)

    @pl.when(i == 0)
    def _():
        ph1_ref[...] = jnp.zeros_like(ph1_ref)

    ph1_ref[...] += jnp.dot(ohT, h, preferred_element_type=jnp.float32)


def _stage2_body(p2_ref, degm_ref, b3_ref, ph1_ref, w2r_ref, b2_ref,
                 wfc_ref, bfc_ref, out_ref, s_scr, cb_scr):
    i = pl.program_id(0)
    p = p2_ref[0] + p2_ref[1]
    aggm = p / degm_ref[...]
    bb = b3_ref[0]
    ohT = (bb == lax.broadcasted_iota(jnp.int32, (N_GRAPHS, ROW_BLK), 0)
           ).astype(jnp.float32)

    @pl.when(i == 0)
    def _():
        s_scr[...] = jnp.zeros_like(s_scr)
        cb_scr[...] = jnp.zeros_like(cb_scr)

    s_scr[...] += jnp.dot(ohT, aggm, preferred_element_type=jnp.float32)
    cb_scr[...] += jnp.dot(ohT, jnp.ones((ROW_BLK, D), jnp.float32),
                           preferred_element_type=jnp.float32)

    @pl.when(i == N_BLKS - 1)
    def _():
        cb = jnp.maximum(cb_scr[:, 0:1], 1.0)
        g = (s_scr[...] / cb
             + jnp.dot(ph1_ref[...] / cb, w2r_ref[...],
                       preferred_element_type=jnp.float32)
             + b2_ref[...])
        out_ref[...] = (jnp.dot(g, wfc_ref[...],
                                preferred_element_type=jnp.float32)
                        + bfc_ref[...])


def kernel(x, edge_index, batch, W1l, b1, W1r, W2l, b2, W2r, Wfc, bfc):
    src = edge_index[0].astype(jnp.int32)
    dst = edge_index[1].astype(jnp.int32)
    epad = E_PAD - N_EDGES
    src_p = jnp.concatenate([src, jnp.zeros((epad,), jnp.int32)]
                            ).reshape(E_PAD // CHUNK, CHUNK)
    dst_p = jnp.concatenate([dst, jnp.full((epad,), DUMMY_ROW, jnp.int32)]
                            ).reshape(E_PAD // CHUNK, CHUNK)
    npad = N_PAD - N_NODES
    x_p = jnp.concatenate([x, jnp.zeros((npad, D), x.dtype)])
    batch3 = jnp.concatenate([batch.astype(jnp.int32),
                              jnp.full((npad,), N_GRAPHS, jnp.int32)]
                             ).reshape(N_BLKS, 1, ROW_BLK)
    zrows = jnp.zeros((CHUNK, D), jnp.float32)

    full = pl.BlockSpec((D, D), lambda i: (0, 0))
    rowb = pl.BlockSpec((ROW_BLK, D), lambda i: (i, 0))

    y1, r1 = pl.pallas_call(
        _mm2_body,
        grid=(N_BLKS,),
        in_specs=[rowb, full, full, pl.BlockSpec((1, D), lambda i: (0, 0))],
        out_specs=[rowb, rowb],
        out_shape=[jax.ShapeDtypeStruct((N_PAD, D), jnp.float32)] * 2,
    )(x_p, W1l, W1r, b1.reshape(1, D))

    p1, deg = _make_edge_pass(True)(y1, src_p, dst_p, zrows)
    degm = jnp.maximum(deg[0] + deg[1], 1.0).reshape(N_PAD, 1)

    pblk = pl.BlockSpec((NC, ROW_BLK, D), lambda i: (0, i, 0))
    dblk = pl.BlockSpec((ROW_BLK, 1), lambda i: (i, 0))
    bblk = pl.BlockSpec((1, 1, ROW_BLK), lambda i: (i, 0, 0))
    gfull = pl.BlockSpec((N_GRAPHS, D), lambda i: (0, 0))

    z, ph1 = pl.pallas_call(
        _stage1_body,
        grid=(N_BLKS,),
        in_specs=[pblk, dblk, rowb, bblk, full],
        out_specs=[rowb, gfull],
        out_shape=[jax.ShapeDtypeStruct((N_PAD, D), jnp.float32),
                   jax.ShapeDtypeStruct((N_GRAPHS, D), jnp.float32)],
    )(p1, degm, r1, batch3, W2l)

    (p2,) = _make_edge_pass(False)(z, src_p, dst_p, zrows)

    out = pl.pallas_call(
        _stage2_body,
        grid=(N_BLKS,),
        in_specs=[pblk, dblk, bblk, gfull, full,
                  pl.BlockSpec((1, D), lambda i: (0, 0)),
                  pl.BlockSpec((D, 4), lambda i: (0, 0)),
                  pl.BlockSpec((1, 4), lambda i: (0, 0))],
        out_specs=pl.BlockSpec((N_GRAPHS, 4), lambda i: (0, 0)),
        out_shape=jax.ShapeDtypeStruct((N_GRAPHS, 4), jnp.float32),
        scratch_shapes=[pltpu.VMEM((N_GRAPHS, D), jnp.float32),
                        pltpu.VMEM((N_GRAPHS, D), jnp.float32)],
    )(p2, degm, batch3, ph1, W2r, b2.reshape(1, D), Wfc, bfc.reshape(1, 4))

    return out
